# Initial kernel scaffold; baseline (speedup 1.0000x reference)
import jax
import jax.numpy as jnp
from jax.experimental import pallas as pl


def kernel(x, edge_index, edge_weight, pred_edge_index, W1, b1, We1, be1, We2, be2, W2, b2, Wp, bp):
    raise NotImplementedError("write your pallas kernel here")


# speedup vs baseline: 1.9102x; 1.9102x over previous
import functools

import jax
import jax.numpy as jnp
from jax import lax
from jax.experimental import pallas as pl
from jax.experimental.pallas import tpu as pltpu
from jax.experimental.pallas import tpu_sc as plsc

N = 10000
N2 = 10240
E = 320000
P = 100000
H = 128
PP = 102400
NC = 2
NS = 16
NW = NC * NS
L = 16
CH = 80
GRP = CH // L
EPT = E // NW
NCH_E = EPT // CH
PPT = PP // NW
NCH_P = PPT // CH
ZR = N2 // NS
ZB = 128
NZB = ZR // ZB

_f32 = jnp.float32
_i32 = jnp.int32


def _mesh():
  return plsc.VectorSubcoreMesh(core_axis_name="c", subcore_axis_name="s",
                                num_cores=NC, num_subcores=NS)


_SC_PARAMS = pltpu.CompilerParams(needs_layout_passes=False)


def _zero_shared(zbuf, shared, s):
  def zrow(i, carry):
    for k in range(H // L):
      zbuf[i, pl.ds(k * L, L)] = jnp.zeros((L,), _f32)
    return carry
  lax.fori_loop(0, ZB, zrow, 0)
  for j in range(NZB):
    pltpu.sync_copy(zbuf, shared.at[pl.ds(s * ZR + j * ZB, ZB)])


def _copy_out(shared, out, c, s):
  for j in range(NZB):
    sl = pl.ds(s * ZR + j * ZB, ZB)
    pltpu.sync_copy(shared.at[sl], out.at[c, sl])


def _scale_rows(rows, wv):
  def erow(e, carry):
    ws = plsc.load_gather(wv, [jnp.full((L,), e, _i32)])
    for k in range(H // L):
      sl = pl.ds(k * L, L)
      rows[e, sl] = rows[e, sl] * ws
    return carry
  lax.fori_loop(0, CH, erow, 0)


def _sc1_body(x_hbm, src_hbm, dst_hbm, w_hbm, out_hbm,
              shared, zbuf, srcv, dstv, wv, rows, sem):
  c = lax.axis_index("c")
  s = lax.axis_index("s")
  base0 = (c * NS + s) * EPT
  _zero_shared(zbuf, shared, s)
  plsc.subcore_barrier()

  def chunk(j, carry):
    base = base0 + j * CH
    pltpu.sync_copy(src_hbm.at[pl.ds(base, CH)], srcv)
    pltpu.sync_copy(dst_hbm.at[pl.ds(base, CH)], dstv)
    pltpu.sync_copy(w_hbm.at[pl.ds(base, CH)], wv)
    pltpu.async_copy(x_hbm.at[srcv], rows, sem).wait()
    _scale_rows(rows, wv)
    pltpu.sync_copy(rows, shared.at[dstv], add=True)
    return carry
  lax.fori_loop(0, NCH_E, chunk, 0)
  plsc.subcore_barrier()
  _copy_out(shared, out_hbm, c, s)


def _sc2_body(ap_hbm, b_hbm, h1_hbm, src_hbm, dst_hbm, w_hbm, we2_hbm, be2_hbm,
              out_hbm, ps_hbm,
              shared, zbuf, srcv, dstv, wv, arows, brows, hrows,
              we2v, be2v, psv, sem):
  c = lax.axis_index("c")
  s = lax.axis_index("s")
  wid = c * NS + s
  base0 = wid * EPT
  pltpu.sync_copy(we2_hbm, we2v)
  pltpu.sync_copy(be2_hbm, be2v)
  _zero_shared(zbuf, shared, s)
  plsc.subcore_barrier()
  iot = lax.iota(_i32, L)
  rows_g = [iot + g * L for g in range(GRP)]
  be2 = be2v[...]

  def chunk(j, psum):
    base = base0 + j * CH
    pltpu.sync_copy(src_hbm.at[pl.ds(base, CH)], srcv)
    pltpu.sync_copy(dst_hbm.at[pl.ds(base, CH)], dstv)
    pltpu.sync_copy(w_hbm.at[pl.ds(base, CH)], wv)
    pltpu.async_copy(ap_hbm.at[srcv], arows, sem).wait()
    pltpu.async_copy(b_hbm.at[dstv], brows, sem).wait()
    pltpu.async_copy(h1_hbm.at[srcv], hrows, sem).wait()

    def kbody(k, accs):
      kf = jnp.full((L,), k, _i32)
      w2 = plsc.load_gather(we2v, [kf])
      return tuple(
          accs[g] + jnp.maximum(
              plsc.load_gather(arows, [rows_g[g], kf])
              + plsc.load_gather(brows, [rows_g[g], kf]), 0.0) * w2
          for g in range(GRP))
    accs = lax.fori_loop(0, H, kbody, (jnp.zeros((L,), _f32),) * GRP)

    for g in range(GRP):
      logit = accs[g] + be2
      prob = 1.0 / (1.0 + jnp.exp(-logit))
      psum = psum + prob
      sl = pl.ds(g * L, L)
      rw = wv[sl] * prob
      rw = jnp.where(jnp.abs(rw) > 0.001, rw, 0.0)
      wv[sl] = rw
    _scale_rows(hrows, wv)
    pltpu.sync_copy(hrows, shared.at[dstv], add=True)
    return psum
  psum = lax.fori_loop(0, NCH_E, chunk, jnp.zeros((L,), _f32))
  psv[...] = psum
  pltpu.sync_copy(psv, ps_hbm.at[pl.ds(wid * L, L)])
  plsc.subcore_barrier()
  _copy_out(shared, out_hbm, c, s)


def _sc3_body(uv_hbm, pu_hbm, pv_hbm, o0_hbm, o1_hbm,
              uvv, puv, pvv, o0v, o1v, sem):
  c = lax.axis_index("c")
  s = lax.axis_index("s")
  base0 = (c * NS + s) * PPT
  pltpu.sync_copy(uv_hbm, uvv)

  def chunk(j, carry):
    base = base0 + j * CH
    pltpu.sync_copy(pu_hbm.at[pl.ds(base, CH)], puv)
    pltpu.sync_copy(pv_hbm.at[pl.ds(base, CH)], pvv)
    for g in range(GRP):
      sl = pl.ds(g * L, L)
      pu4 = puv[sl] * 4
      pv4 = pvv[sl] * 4
      o0v[sl] = (plsc.load_gather(uvv, [pu4])
                 + plsc.load_gather(uvv, [pv4 + 2]))
      o1v[sl] = (plsc.load_gather(uvv, [pu4 + 1])
                 + plsc.load_gather(uvv, [pv4 + 3]))
    pltpu.sync_copy(o0v, o0_hbm.at[pl.ds(base, CH)])
    pltpu.sync_copy(o1v, o1_hbm.at[pl.ds(base, CH)])
    return carry
  lax.fori_loop(0, NCH_P, chunk, 0)


_sc1 = functools.partial(
    pl.kernel,
    out_type=jax.ShapeDtypeStruct((NC, N2, H), _f32),
    mesh=_mesh(),
    compiler_params=_SC_PARAMS,
    scratch_types=[
        pltpu.VMEM_SHARED((N2, H), _f32),
        pltpu.VMEM((ZB, H), _f32),
        pltpu.VMEM((CH,), _i32),
        pltpu.VMEM((CH,), _i32),
        pltpu.VMEM((CH,), _f32),
        pltpu.VMEM((CH, H), _f32),
        pltpu.SemaphoreType.DMA,
    ])(_sc1_body)

_sc2 = functools.partial(
    pl.kernel,
    out_type=(jax.ShapeDtypeStruct((NC, N2, H), _f32),
              jax.ShapeDtypeStruct((NW * L,), _f32)),
    mesh=_mesh(),
    compiler_params=_SC_PARAMS,
    scratch_types=[
        pltpu.VMEM_SHARED((N2, H), _f32),
        pltpu.VMEM((ZB, H), _f32),
        pltpu.VMEM((CH,), _i32),
        pltpu.VMEM((CH,), _i32),
        pltpu.VMEM((CH,), _f32),
        pltpu.VMEM((CH, H), _f32),
        pltpu.VMEM((CH, H), _f32),
        pltpu.VMEM((CH, H), _f32),
        pltpu.VMEM((H,), _f32),
        pltpu.VMEM((L,), _f32),
        pltpu.VMEM((L,), _f32),
        pltpu.SemaphoreType.DMA,
    ])(_sc2_body)

_sc3 = functools.partial(
    pl.kernel,
    out_type=(jax.ShapeDtypeStruct((PP,), _f32),
              jax.ShapeDtypeStruct((PP,), _f32)),
    mesh=_mesh(),
    compiler_params=_SC_PARAMS,
    scratch_types=[
        pltpu.VMEM((N2 * 4,), _f32),
        pltpu.VMEM((CH,), _i32),
        pltpu.VMEM((CH,), _i32),
        pltpu.VMEM((CH,), _f32),
        pltpu.VMEM((CH,), _f32),
        pltpu.SemaphoreType.DMA,
    ])(_sc3_body)


RB = 2048
NRB = N2 // RB


def _tc1_body(x_ref, a_ref, b_ref, w1_ref, b1_ref, wt_ref, wb_ref, be1_ref,
              h1_ref, ap_ref, bm_ref):
  h = x_ref[...] + a_ref[...] + b_ref[...]
  h1 = jnp.maximum(
      jnp.dot(h, w1_ref[...], preferred_element_type=_f32) + b1_ref[...], 0.0)
  h1_ref[...] = h1
  ap_ref[...] = jnp.dot(h1, wt_ref[...], preferred_element_type=_f32) + be1_ref[...]
  bm_ref[...] = jnp.dot(h1, wb_ref[...], preferred_element_type=_f32)


def _tc2_body(h1_ref, a_ref, b_ref, w2_ref, b2_ref, wt_ref, wb_ref, bp_ref,
              ps_ref, uv_ref, l1_ref):
  h1 = h1_ref[...]
  h2 = jnp.maximum(
      jnp.dot(h1 + a_ref[...] + b_ref[...], w2_ref[...],
              preferred_element_type=_f32) + b2_ref[...], 0.0)
  uv_ref[...] = (jnp.dot(h1, wt_ref[...], preferred_element_type=_f32)
                 + jnp.dot(h2, wb_ref[...], preferred_element_type=_f32)
                 + bp_ref[...])

  @pl.when(pl.program_id(0) == 0)
  def _():
    l1_ref[...] = jnp.reshape(jnp.sum(ps_ref[...]) / float(E), (1, 1))


def _row_spec(r, cols):
  return pl.BlockSpec((r, cols), lambda i: (i, 0))


def _full_spec(rows, cols):
  return pl.BlockSpec((rows, cols), lambda i: (0, 0))


def _tc1(x, a0, a1, w1, b1, wt, wb, be1):
  return pl.pallas_call(
      _tc1_body,
      grid=(NRB,),
      in_specs=[_row_spec(RB, H), _row_spec(RB, H), _row_spec(RB, H),
                _full_spec(H, H), _full_spec(1, H),
                _full_spec(H, H), _full_spec(H, H), _full_spec(1, H)],
      out_specs=[_row_spec(RB, H), _row_spec(RB, H), _row_spec(RB, H)],
      out_shape=[jax.ShapeDtypeStruct((N2, H), _f32)] * 3,
  )(x, a0, a1, w1, b1, wt, wb, be1)


def _tc2(h1, a0, a1, w2, b2, wt, wb, bpuv, ps):
  return pl.pallas_call(
      _tc2_body,
      grid=(NRB,),
      in_specs=[_row_spec(RB, H), _row_spec(RB, H), _row_spec(RB, H),
                _full_spec(H, H), _full_spec(1, H),
                _full_spec(H, 4), _full_spec(H, 4), _full_spec(1, 4),
                _full_spec(1, NW * L)],
      out_specs=[_row_spec(RB, 4), _full_spec(1, 1)],
      out_shape=[jax.ShapeDtypeStruct((N2, 4), _f32),
                 jax.ShapeDtypeStruct((1, 1), _f32)],
  )(h1, a0, a1, w2, b2, wt, wb, bpuv, ps)


def kernel(x, edge_index, edge_weight, pred_edge_index,
           W1, b1, We1, be1, We2, be2, W2, b2, Wp, bp):
  src = edge_index[0].astype(_i32)
  dst = edge_index[1].astype(_i32)
  w = edge_weight.astype(_f32)
  pad = PP - P
  pu = jnp.concatenate([pred_edge_index[0].astype(_i32),
                        jnp.zeros((pad,), _i32)])
  pv = jnp.concatenate([pred_edge_index[1].astype(_i32),
                        jnp.zeros((pad,), _i32)])

  x2 = jnp.pad(x, ((0, N2 - N), (0, 0)))
  agg1 = _sc1(x2, src, dst, w)
  h1, ap, bm = _tc1(x2, agg1[0], agg1[1], W1, b1.reshape(1, H),
                    We1[:H], We1[H:], be1.reshape(1, H))
  agg2, ps = _sc2(ap, bm, h1, src, dst, w, We2[:, 0],
                  jnp.full((L,), be2[0], _f32))
  wuvt = jnp.concatenate([Wp[0:H], Wp[2 * H:3 * H]], axis=1)
  wuvb = jnp.concatenate([Wp[H:2 * H], Wp[3 * H:4 * H]], axis=1)
  bpuv = jnp.concatenate([bp, jnp.zeros_like(bp)]).reshape(1, 4)
  uv, l1 = _tc2(h1, agg2[0], agg2[1], W2, b2.reshape(1, H),
                wuvt, wuvb, bpuv, ps.reshape(1, NW * L))
  o0, o1 = _sc3(uv.reshape(N2 * 4), pu, pv)
  edge_logits = jnp.stack([o0[:P], o1[:P]], axis=1)
  return edge_logits, l1[0, 0]
